# baseline (speedup 1.0000x reference)
import functools

import jax
import jax.numpy as jnp
from jax import lax
from jax.experimental import pallas as pl
from jax.experimental.pallas import tpu as pltpu
from jax.experimental.pallas import tpu_sc as plsc

VOCAB = 100000
EMBED_DIM = 128
BATCH = 16384
SEQ = 200

VB = 8192
VP = ((VOCAB + 8 + VB - 1) // VB) * VB
BIAS_IDX = VOCAB
VR = VOCAB + 8

NC = 2
NS = 16
NW = NC * NS
RPT = BATCH // NW
CH = 32
NCH = RPT // CH
CHW = CH * SEQ
NFULL = SEQ // 16
TAIL = SEQ - NFULL * 16


def _proj_body(w_ref, pb_ref, emb_ref, o_ref):
    i = pl.program_id(0)
    acc = lax.dot_general(
        w_ref[...], emb_ref[...],
        (((1,), (1,)), ((), ())),
        preferred_element_type=jnp.float32,
    )
    u0 = lax.bitcast_convert_type(
        acc[0:1, :].astype(jnp.bfloat16), jnp.uint16).astype(jnp.int32)
    u1 = lax.bitcast_convert_type(
        acc[1:2, :].astype(jnp.bfloat16), jnp.uint16).astype(jnp.int32)
    packed = u0 | (u1 << 16)
    v = i * VB + lax.broadcasted_iota(jnp.int32, (1, VB), 1)
    o_ref[...] = jnp.where(v < VOCAB, packed,
                           jnp.where(v == BIAS_IDX, pb_ref[...], 0))


def _project_table(emb_table, fc_w, fc_b):
    w8 = jnp.zeros((8, EMBED_DIM), jnp.float32).at[:2].set(fc_w * (1.0 / SEQ))
    bu = lax.bitcast_convert_type(
        fc_b.astype(jnp.bfloat16), jnp.uint16).astype(jnp.int32)
    pbias = bu[0] | (bu[1] << 16)
    pb_full = jnp.broadcast_to(pbias, (1, VB))
    return pl.pallas_call(
        _proj_body,
        grid=(VP // VB,),
        in_specs=[
            pl.BlockSpec((8, EMBED_DIM), lambda i: (0, 0)),
            pl.BlockSpec((1, VB), lambda i: (0, 0)),
            pl.BlockSpec((VB, EMBED_DIM), lambda i: (i, 0)),
        ],
        out_specs=pl.BlockSpec((1, VB), lambda i: (0, i)),
        out_shape=jax.ShapeDtypeStruct((1, VP), jnp.int32),
    )(w8, pb_full, emb_table).reshape(VP)


def _sc_pool_body(p_hbm, x_hbm, out_hbm,
                  p_v, xa_v, xb_v, o_v, t0_v, t1_v, sem_p, sem_a, sem_b):
    wid = lax.axis_index("s") * NC + lax.axis_index("c")
    row0 = wid * RPT

    cp_p = pltpu.async_copy(p_hbm.at[pl.ds(0, VR)], p_v, sem_p)

    bufs = (xa_v, xb_v)
    sems = (sem_a, sem_b)
    handles = [None] * NCH
    handles[0] = pltpu.async_copy(
        x_hbm.at[pl.ds(row0, CH)], xa_v, sem_a)

    cp_p.wait()
    bvec = plsc.load_gather(p_v, [jnp.full((16,), BIAS_IDX, jnp.int32)])
    b0s, b1s = plsc.unpack(plsc.bitcast(bvec, jnp.bfloat16),
                           format=plsc.PackFormat.INTERLEAVED)
    lane = lax.broadcasted_iota(jnp.int32, (16,), 0)
    tail_mask = lane >= (16 - TAIL)
    zf = jnp.zeros((16,), jnp.float32)

    for ch in range(NCH):
        if ch + 1 < NCH:
            handles[ch + 1] = pltpu.async_copy(
                x_hbm.at[pl.ds(row0 + (ch + 1) * CH, CH)],
                bufs[(ch + 1) % 2], sems[(ch + 1) % 2])
        handles[ch].wait()
        cur = bufs[ch % 2]

        def group_body(g, carry, cur=cur, ch=ch):
            @plsc.parallel_loop(0, 16, unroll=2)
            def row_body(rr):
                row = g * 16 + rr
                acc0 = zf
                acc1 = zf
                for j in range(NFULL):
                    gi = plsc.load_gather(p_v, [cur[row, pl.ds(j * 16, 16)]])
                    a, b = plsc.unpack(plsc.bitcast(gi, jnp.bfloat16),
                                       format=plsc.PackFormat.INTERLEAVED)
                    acc0 = acc0 + a
                    acc1 = acc1 + b
                gi = plsc.load_gather(p_v, [cur[row, pl.ds(SEQ - 16, 16)]])
                a, b = plsc.unpack(plsc.bitcast(gi, jnp.bfloat16),
                                   format=plsc.PackFormat.INTERLEAVED)
                acc0 = acc0 + jnp.where(tail_mask, a, 0.0)
                acc1 = acc1 + jnp.where(tail_mask, b, 0.0)
                t0_v[pl.ds(rr * 16, 16)] = acc0
                t1_v[pl.ds(rr * 16, 16)] = acc1

            col = lane * 16
            s0 = zf
            s1 = zf
            for c in range(16):
                s0 = s0 + plsc.load_gather(t0_v, [col + c])
                s1 = s1 + plsc.load_gather(t1_v, [col + c])
            base = ch * CH + g * 16
            o_v[pl.ds(base, 16)] = s0 + b0s
            o_v[pl.ds(RPT + base, 16)] = s1 + b1s
            return carry

        lax.fori_loop(0, CH // 16, group_body, 0)

    pltpu.sync_copy(o_v.at[pl.ds(0, RPT)], out_hbm.at[0, pl.ds(row0, RPT)])
    pltpu.sync_copy(o_v.at[pl.ds(RPT, RPT)], out_hbm.at[1, pl.ds(row0, RPT)])


_sc_pool = functools.partial(
    pl.kernel,
    out_type=jax.ShapeDtypeStruct((2, BATCH), jnp.float32),
    mesh=plsc.VectorSubcoreMesh(
        core_axis_name="c", subcore_axis_name="s",
        num_cores=NC, num_subcores=NS,
    ),
    scratch_types=[
        pltpu.VMEM((VR,), jnp.int32),
        pltpu.VMEM((CH, SEQ), jnp.int32),
        pltpu.VMEM((CH, SEQ), jnp.int32),
        pltpu.VMEM((2 * RPT,), jnp.float32),
        pltpu.VMEM((256,), jnp.float32),
        pltpu.VMEM((256,), jnp.float32),
        pltpu.SemaphoreType.DMA,
        pltpu.SemaphoreType.DMA,
        pltpu.SemaphoreType.DMA,
    ],
    compiler_params=pltpu.CompilerParams(needs_layout_passes=False),
)(_sc_pool_body)


@jax.jit
def kernel(x, emb_table, fc_w, fc_b):
    p_packed = _project_table(emb_table, fc_w, fc_b)
    out2 = _sc_pool(p_packed, x.astype(jnp.int32))
    return out2.T


# speedup vs baseline: 1.0633x; 1.0633x over previous
import functools

import jax
import jax.numpy as jnp
from jax import lax
from jax.experimental import pallas as pl
from jax.experimental.pallas import tpu as pltpu
from jax.experimental.pallas import tpu_sc as plsc

VOCAB = 100000
EMBED_DIM = 128
BATCH = 16384
SEQ = 200

VB = 12800
VP = ((VOCAB + 8 + VB - 1) // VB) * VB
BIAS_IDX = VOCAB
VR = VOCAB + 8

NC = 2
NS = 16
NW = NC * NS
RPT = BATCH // NW
CH = 32
NCH = RPT // CH
CHW = CH * SEQ
NFULL = SEQ // 16
TAIL = SEQ - NFULL * 16


def _proj_body(w_ref, pb_ref, emb_ref, o_ref):
    i = pl.program_id(0)
    acc = lax.dot_general(
        w_ref[...], emb_ref[...],
        (((1,), (1,)), ((), ())),
        preferred_element_type=jnp.float32,
    )
    u0 = lax.bitcast_convert_type(
        acc[0:1, :].astype(jnp.bfloat16), jnp.uint16).astype(jnp.int32)
    u1 = lax.bitcast_convert_type(
        acc[1:2, :].astype(jnp.bfloat16), jnp.uint16).astype(jnp.int32)
    packed = u0 | (u1 << 16)
    v = i * VB + lax.broadcasted_iota(jnp.int32, (1, VB), 1)
    o_ref[...] = jnp.where(v < VOCAB, packed,
                           jnp.where(v == BIAS_IDX, pb_ref[...], 0))


def _project_table(emb_table, fc_w, fc_b):
    w8 = jnp.zeros((8, EMBED_DIM), jnp.float32).at[:2].set(fc_w * (1.0 / SEQ))
    bu = lax.bitcast_convert_type(
        fc_b.astype(jnp.bfloat16), jnp.uint16).astype(jnp.int32)
    pbias = bu[0] | (bu[1] << 16)
    pb_full = jnp.broadcast_to(pbias, (1, VB))
    return pl.pallas_call(
        _proj_body,
        grid=(VP // VB,),
        in_specs=[
            pl.BlockSpec((8, EMBED_DIM), lambda i: (0, 0)),
            pl.BlockSpec((1, VB), lambda i: (0, 0)),
            pl.BlockSpec((VB, EMBED_DIM), lambda i: (i, 0)),
        ],
        out_specs=pl.BlockSpec((1, VB), lambda i: (0, i)),
        out_shape=jax.ShapeDtypeStruct((1, VP), jnp.int32),
    )(w8, pb_full, emb_table).reshape(VP)


def _sc_pool_body(p_hbm, x_hbm, out_hbm,
                  p_v, xa_v, xb_v, o_v, sem_p, sem_a, sem_b):
    wid = lax.axis_index("s") * NC + lax.axis_index("c")
    row0 = wid * RPT

    cp_p = pltpu.async_copy(p_hbm.at[pl.ds(0, VR)], p_v, sem_p)

    bufs = (xa_v, xb_v)
    sems = (sem_a, sem_b)
    handles = [None] * NCH
    handles[0] = pltpu.async_copy(
        x_hbm.at[pl.ds(row0, CH)], xa_v, sem_a)

    cp_p.wait()
    bvec = plsc.load_gather(p_v, [jnp.full((16,), BIAS_IDX, jnp.int32)])
    b0s, b1s = plsc.unpack(plsc.bitcast(bvec, jnp.bfloat16),
                           format=plsc.PackFormat.INTERLEAVED)
    lane = lax.broadcasted_iota(jnp.int32, (16,), 0)
    tail_mask = lane >= (16 - TAIL)
    zf = jnp.zeros((16,), jnp.float32)

    for ch in range(NCH):
        if ch + 1 < NCH:
            handles[ch + 1] = pltpu.async_copy(
                x_hbm.at[pl.ds(row0 + (ch + 1) * CH, CH)],
                bufs[(ch + 1) % 2], sems[(ch + 1) % 2])
        handles[ch].wait()
        cur = bufs[ch % 2]

        def group_body(g, carry, cur=cur, ch=ch):
            def row_body(rr, vecs):
                vec0, vec1 = vecs
                row = g * 16 + rr
                acc0 = zf
                acc1 = zf
                for j in range(NFULL):
                    gi = plsc.load_gather(p_v, [cur[row, pl.ds(j * 16, 16)]])
                    a, b = plsc.unpack(plsc.bitcast(gi, jnp.bfloat16),
                                       format=plsc.PackFormat.INTERLEAVED)
                    acc0 = acc0 + a
                    acc1 = acc1 + b
                gi = plsc.load_gather(p_v, [cur[row, pl.ds(SEQ - 16, 16)]])
                a, b = plsc.unpack(plsc.bitcast(gi, jnp.bfloat16),
                                   format=plsc.PackFormat.INTERLEAVED)
                acc0 = acc0 + jnp.where(tail_mask, a, 0.0)
                acc1 = acc1 + jnp.where(tail_mask, b, 0.0)
                return (jnp.where(lane == rr, jnp.sum(acc0), vec0),
                        jnp.where(lane == rr, jnp.sum(acc1), vec1))

            vec0, vec1 = lax.fori_loop(0, 16, row_body, (zf, zf))
            base = ch * CH + g * 16
            o_v[pl.ds(base, 16)] = vec0 + b0s
            o_v[pl.ds(RPT + base, 16)] = vec1 + b1s
            return carry

        lax.fori_loop(0, CH // 16, group_body, 0)

    pltpu.sync_copy(o_v.at[pl.ds(0, RPT)], out_hbm.at[0, pl.ds(row0, RPT)])
    pltpu.sync_copy(o_v.at[pl.ds(RPT, RPT)], out_hbm.at[1, pl.ds(row0, RPT)])


_sc_pool = functools.partial(
    pl.kernel,
    out_type=jax.ShapeDtypeStruct((2, BATCH), jnp.float32),
    mesh=plsc.VectorSubcoreMesh(
        core_axis_name="c", subcore_axis_name="s",
        num_cores=NC, num_subcores=NS,
    ),
    scratch_types=[
        pltpu.VMEM((VR,), jnp.int32),
        pltpu.VMEM((CH, SEQ), jnp.int32),
        pltpu.VMEM((CH, SEQ), jnp.int32),
        pltpu.VMEM((2 * RPT,), jnp.float32),
        pltpu.SemaphoreType.DMA,
        pltpu.SemaphoreType.DMA,
        pltpu.SemaphoreType.DMA,
    ],
    compiler_params=pltpu.CompilerParams(needs_layout_passes=False),
)(_sc_pool_body)


@jax.jit
def kernel(x, emb_table, fc_w, fc_b):
    p_packed = _project_table(emb_table, fc_w, fc_b)
    out2 = _sc_pool(p_packed, x.astype(jnp.int32))
    return out2.T
